# Initial kernel scaffold; baseline (speedup 1.0000x reference)
import jax
import jax.numpy as jnp
from jax.experimental import pallas as pl


def kernel(feat1, feat2, prob1, prob2):
    raise NotImplementedError("write your pallas kernel here")


# speedup vs baseline: 53.6109x; 53.6109x over previous
import jax
import jax.numpy as jnp
from jax.experimental import pallas as pl
from jax.experimental.pallas import tpu as pltpu

_N = 256
_D = 256
_TOPK = 5


def _rank_loss_kernel(feat1_ref, prob1_ref, prob2_ref, out_ref):
    feat = feat1_ref[...]
    col = jax.lax.broadcasted_iota(jnp.int32, (_N, _D), 1)
    vals = feat
    mask = jnp.zeros((_N, _D), jnp.float32)
    for _ in range(_TOPK):
        m = jnp.max(vals, axis=1, keepdims=True)
        sel = jnp.min(jnp.where(vals == m, col, _D), axis=1, keepdims=True)
        hit = col == sel
        mask = jnp.where(hit, 1.0, mask)
        vals = jnp.where(hit, -jnp.inf, vals)

    overlap = jax.lax.dot_general(
        mask, mask, (((1,), (1,)), ((), ())),
        preferred_element_type=jnp.float32)
    target = overlap > (_TOPK - 0.5)

    sim = jax.lax.dot_general(
        prob2_ref[...], prob1_ref[...], (((1,), (1,)), ((), ())),
        precision=jax.lax.Precision.HIGHEST,
        preferred_element_type=jnp.float32)
    eps = 1e-12
    p = jnp.clip(sim, eps, 1.0 - eps)
    terms = jnp.where(target, jnp.log(p), jnp.log1p(-p))
    out_ref[0, 0] = -jnp.sum(terms) / (_N * _N)


def kernel(feat1, feat2, prob1, prob2):
    del feat2
    out = pl.pallas_call(
        _rank_loss_kernel,
        out_shape=jax.ShapeDtypeStruct((1, 1), jnp.float32),
        out_specs=pl.BlockSpec(memory_space=pltpu.SMEM),
    )(feat1, prob1, prob2)
    return out.reshape(())


# speedup vs baseline: 66.5606x; 1.2415x over previous
import jax
import jax.numpy as jnp
from jax.experimental import pallas as pl
from jax.experimental.pallas import tpu as pltpu

_N = 256
_D = 256
_TOPK = 5


def _rank_loss_kernel(feat1_ref, prob1_ref, prob2_ref, out_ref):
    feat = feat1_ref[...]
    colf = jax.lax.broadcasted_iota(jnp.int32, (_N, _D), 1).astype(jnp.float32)
    vals = feat
    mask = jnp.zeros((_N, _D), jnp.float32)
    for _ in range(_TOPK):
        m = jnp.max(vals, axis=1, keepdims=True)
        sel = jnp.min(jnp.where(vals == m, colf, float(_D)),
                      axis=1, keepdims=True)
        hit = colf == sel
        mask = jnp.where(hit, 1.0, mask)
        vals = jnp.where(hit, -jnp.inf, vals)

    mask_bf = mask.astype(jnp.bfloat16)
    overlap = jax.lax.dot_general(
        mask_bf, mask_bf, (((1,), (1,)), ((), ())),
        preferred_element_type=jnp.float32)
    target = overlap > (_TOPK - 0.5)

    sim = jax.lax.dot_general(
        prob2_ref[...].astype(jnp.bfloat16),
        prob1_ref[...].astype(jnp.bfloat16),
        (((1,), (1,)), ((), ())),
        preferred_element_type=jnp.float32)
    eps = 1e-12
    p = jnp.clip(sim, eps, 1.0 - eps)
    q = jnp.where(target, p, 1.0 - p)
    out_ref[0, 0] = -jnp.sum(jnp.log(q)) / (_N * _N)


def kernel(feat1, feat2, prob1, prob2):
    del feat2
    out = pl.pallas_call(
        _rank_loss_kernel,
        out_shape=jax.ShapeDtypeStruct((1, 1), jnp.float32),
        out_specs=pl.BlockSpec(memory_space=pltpu.SMEM),
    )(feat1, prob1, prob2)
    return out.reshape(())
